# Initial kernel scaffold; baseline (speedup 1.0000x reference)
import jax
import jax.numpy as jnp
from jax.experimental import pallas as pl


def kernel(x, W_enc, b_enc, W_dec, b_pre, model_idx):
    raise NotImplementedError("write your pallas kernel here")


# speedup vs baseline: 12.2269x; 12.2269x over previous
import functools

import jax
import jax.numpy as jnp
from jax.experimental import pallas as pl
from jax.experimental.pallas import tpu as pltpu

_K = 32
_BM = 256

_INT32_MIN = -2147483648


def _sae_block_kernel(x_ref, we_ref, be_ref, wd_ref, bp_ref, o_ref):
    bp = bp_ref[...]
    xc = x_ref[...] - bp
    pre = jax.lax.dot_general(
        xc, we_ref[...], (((1,), (1,)), ((), ())),
        preferred_element_type=jnp.float32,
        precision=jax.lax.Precision.DEFAULT,
    ) + be_ref[...]

    i = jax.lax.bitcast_convert_type(pre, jnp.int32)
    s = i ^ ((i >> 31) & 0x7FFFFFFF)

    def body(b, tu):
        bit = 31 - b
        cand = tu | (jnp.int32(1) << bit)
        thr = cand ^ _INT32_MIN
        cnt = jnp.sum((s >= thr).astype(jnp.int32), axis=1, keepdims=True)
        return jnp.where(cnt >= _K, cand, tu)

    tu0 = jnp.zeros((s.shape[0], 1), jnp.int32)
    tu = jax.lax.fori_loop(0, 32, body, tu0)
    t = tu ^ _INT32_MIN

    z = jnp.where(s >= t, pre, 0.0)
    rec = jax.lax.dot_general(
        z, wd_ref[...], (((1,), (1,)), ((), ())),
        preferred_element_type=jnp.float32,
        precision=jax.lax.Precision.HIGHEST,
    )
    o_ref[...] = rec + bp


def kernel(x, W_enc, b_enc, W_dec, b_pre, model_idx):
    n, d = x.shape
    latent = W_enc.shape[0]
    assert n % _BM == 0
    be2 = b_enc.reshape(1, latent)
    bp2 = b_pre.reshape(1, d)
    return pl.pallas_call(
        _sae_block_kernel,
        grid=(n // _BM,),
        in_specs=[
            pl.BlockSpec((_BM, d), lambda i: (i, 0)),
            pl.BlockSpec((latent, d), lambda i: (0, 0)),
            pl.BlockSpec((1, latent), lambda i: (0, 0)),
            pl.BlockSpec((d, latent), lambda i: (0, 0)),
            pl.BlockSpec((1, d), lambda i: (0, 0)),
        ],
        out_specs=pl.BlockSpec((_BM, d), lambda i: (i, 0)),
        out_shape=jax.ShapeDtypeStruct((n, d), jnp.float32),
        compiler_params=pltpu.CompilerParams(
            dimension_semantics=("parallel",),
        ),
    )(x, W_enc, be2, W_dec, bp2)


# speedup vs baseline: 18.1161x; 1.4817x over previous
import functools

import jax
import jax.numpy as jnp
from jax.experimental import pallas as pl
from jax.experimental.pallas import tpu as pltpu

_K = 32
_BM = 256

_INT32_MIN = -2147483648


def _sae_block_kernel(x_ref, we_ref, be_ref, wd_ref, bp_ref, o_ref):
    bp = bp_ref[...]
    xc = x_ref[...] - bp
    pre = jax.lax.dot_general(
        xc, we_ref[...], (((1,), (1,)), ((), ())),
        preferred_element_type=jnp.float32,
        precision=jax.lax.Precision.DEFAULT,
    ) + be_ref[...]

    i = jax.lax.bitcast_convert_type(pre, jnp.int32)
    s = i ^ ((i >> 31) & 0x7FFFFFFF)

    def body(b, tu):
        bit = 31 - b
        cand = tu | (jnp.int32(1) << bit)
        thr = cand ^ _INT32_MIN
        cnt = jnp.sum((s >= thr).astype(jnp.int32), axis=1, keepdims=True)
        return jnp.where(cnt >= _K, cand, tu)

    tu0 = jnp.zeros((s.shape[0], 1), jnp.int32)
    tu = jax.lax.fori_loop(0, 32, body, tu0)
    t = tu ^ _INT32_MIN

    z = jnp.where(s >= t, pre, 0.0)
    rec = jax.lax.dot_general(
        z, wd_ref[...], (((1,), (1,)), ((), ())),
        preferred_element_type=jnp.float32,
        precision=jax.lax.Precision.DEFAULT,
    )
    o_ref[...] = rec + bp


def kernel(x, W_enc, b_enc, W_dec, b_pre, model_idx):
    n, d = x.shape
    latent = W_enc.shape[0]
    assert n % _BM == 0
    be2 = b_enc.reshape(1, latent)
    bp2 = b_pre.reshape(1, d)
    return pl.pallas_call(
        _sae_block_kernel,
        grid=(n // _BM,),
        in_specs=[
            pl.BlockSpec((_BM, d), lambda i: (i, 0)),
            pl.BlockSpec((latent, d), lambda i: (0, 0)),
            pl.BlockSpec((1, latent), lambda i: (0, 0)),
            pl.BlockSpec((d, latent), lambda i: (0, 0)),
            pl.BlockSpec((1, d), lambda i: (0, 0)),
        ],
        out_specs=pl.BlockSpec((_BM, d), lambda i: (i, 0)),
        out_shape=jax.ShapeDtypeStruct((n, d), jnp.float32),
        compiler_params=pltpu.CompilerParams(
            dimension_semantics=("parallel",),
        ),
    )(x, W_enc, be2, W_dec, bp2)
